# Initial kernel scaffold; baseline (speedup 1.0000x reference)
import jax
import jax.numpy as jnp
from jax.experimental import pallas as pl


def kernel(x, skip, edge_index, kid_trans, kid_up, kid1, kid2, kid3, W_trans, W_up, W1, W2, W3, g_tbn, b_tbn, g1, b1, g2, b2, g3, b3):
    raise NotImplementedError("write your pallas kernel here")


# speedup vs baseline: 1.2649x; 1.2649x over previous
import functools

import jax
import jax.numpy as jnp
from jax import lax
from jax.experimental import pallas as pl
from jax.experimental.pallas import tpu as pltpu
from jax.experimental.pallas import tpu_sc as plsc

N = 10000
E = 160000
C = 128
K_OFF = 27
EPS = 1e-5
SLOPE = 0.01

CH = 128
EP = 163840
NCHUNK = EP // CH
NTILES = 32
CPT = NCHUNK // NTILES
NROWS = 10240
ZSLAB = NROWS // 16
WSLAB = 632


def _mm_body(x_ref, w_ref, o_ref):
    o_ref[0] = jnp.dot(x_ref[...], w_ref[0], preferred_element_type=jnp.float32)


def _per_offset_transform(feat, W):
    nblk = 10
    bn = N // nblk
    return pl.pallas_call(
        _mm_body,
        grid=(nblk, K_OFF),
        in_specs=[
            pl.BlockSpec((bn, C), lambda n, k: (n, 0)),
            pl.BlockSpec((1, C, C), lambda n, k: (k, 0, 0)),
        ],
        out_specs=pl.BlockSpec((1, bn, C), lambda n, k: (k, n, 0)),
        out_shape=jax.ShapeDtypeStruct((K_OFF, N, C), jnp.float32),
    )(feat, W)


def _edge_accum_body(y_hbm, src_hbm, dst_hbm, kid_hbm, out_hbm,
                     srcb, kidb, dstb, gidx, rows, accum, sem):
    cid = lax.axis_index("c")
    sid = lax.axis_index("s")
    wid = cid * 16 + sid

    if True:
        def _zero_row(r, _):
            for j in range(8):
                rows[r, pl.ds(j * 16, 16)] = jnp.zeros((16,), jnp.float32)
            return _
        lax.fori_loop(0, CH, _zero_row, 0)
        for b in range(ZSLAB // CH):
            pltpu.sync_copy(rows, accum.at[pl.ds(sid * ZSLAB + b * CH, CH)])
        plsc.subcore_barrier()

        base = wid * CPT
        pltpu.sync_copy(src_hbm.at[pl.ds(base, CPT)], srcb)
        pltpu.sync_copy(kid_hbm.at[pl.ds(base, CPT)], kidb)
        pltpu.sync_copy(dst_hbm.at[pl.ds(base, CPT)], dstb)

        def _flat_row(c, _):
            for j in range(8):
                s = pl.ds(j * 16, 16)
                gidx[c, s] = kidb[c, s] * N + srcb[c, s]
            return _
        lax.fori_loop(0, CPT, _flat_row, 0)

        def _chunk(c, _):
            pltpu.async_copy(y_hbm.at[gidx.at[c]], rows, sem).wait()
            pltpu.sync_copy(rows, accum.at[dstb.at[c]], add=True)
            return _
        lax.fori_loop(0, CPT, _chunk, 0)
        plsc.subcore_barrier()

        @pl.when(sid < 15)
        def _full_slab():
            pltpu.sync_copy(accum.at[pl.ds(sid * WSLAB, WSLAB)],
                            out_hbm.at[cid, pl.ds(sid * WSLAB, WSLAB)])

        @pl.when(sid == 15)
        def _tail_slab():
            pltpu.sync_copy(accum.at[pl.ds(15 * WSLAB, N - 15 * WSLAB)],
                            out_hbm.at[cid, pl.ds(15 * WSLAB, N - 15 * WSLAB)])


@functools.partial(
    pl.kernel,
    out_type=jax.ShapeDtypeStruct((2, N, C), jnp.float32),
    mesh=plsc.VectorSubcoreMesh(core_axis_name="c", subcore_axis_name="s"),
    scratch_types=[
        pltpu.VMEM((CPT, CH), jnp.int32),
        pltpu.VMEM((CPT, CH), jnp.int32),
        pltpu.VMEM((CPT, CH), jnp.int32),
        pltpu.VMEM((CPT, CH), jnp.int32),
        pltpu.VMEM((CH, C), jnp.float32),
        pltpu.VMEM_SHARED((NROWS, C), jnp.float32),
        pltpu.SemaphoreType.DMA,
    ],
)
def _edge_accum(y_hbm, src_hbm, dst_hbm, kid_hbm, out_hbm,
                srcb, kidb, dstb, gidx, rows, accum, sem):
    _edge_accum_body(y_hbm, src_hbm, dst_hbm, kid_hbm, out_hbm,
                     srcb, kidb, dstb, gidx, rows, accum, sem)


def _bn1_body(p_ref, g_ref, b_ref, o_ref):
    h = p_ref[0] + p_ref[1]
    a = jnp.where(h >= 0, h, SLOPE * h)
    m = jnp.mean(a, axis=0, keepdims=True)
    d = a - m
    v = jnp.mean(d * d, axis=0, keepdims=True)
    o_ref[...] = g_ref[...] * d * lax.rsqrt(v + EPS) + b_ref[...]


def _lrelu_bn(partials, g, b):
    return pl.pallas_call(
        _bn1_body,
        out_shape=jax.ShapeDtypeStruct((N, C), jnp.float32),
    )(partials, g.reshape(1, C), b.reshape(1, C))


def _bn2_body(p_ref, s_ref, g_ref, b_ref, o_ref):
    h = p_ref[0] + p_ref[1] + s_ref[...]
    m = jnp.mean(h, axis=0, keepdims=True)
    d = h - m
    v = jnp.mean(d * d, axis=0, keepdims=True)
    o_ref[...] = g_ref[...] * d * lax.rsqrt(v + EPS) + b_ref[...]


def _skip_bn(partials, skip, g, b):
    return pl.pallas_call(
        _bn2_body,
        out_shape=jax.ShapeDtypeStruct((N, C), jnp.float32),
    )(partials, skip, g.reshape(1, C), b.reshape(1, C))


def kernel(x, skip, edge_index, kid_trans, kid_up, kid1, kid2, kid3,
           W_trans, W_up, W1, W2, W3,
           g_tbn, b_tbn, g1, b1, g2, b2, g3, b3):
    pad = EP - E
    src = jnp.concatenate([edge_index[0], jnp.zeros((pad,), jnp.int32)])
    dst = jnp.concatenate([edge_index[1], jnp.full((pad,), N, jnp.int32)])
    kt = jnp.concatenate([kid_trans, jnp.zeros((pad,), jnp.int32)])
    ku = jnp.concatenate([kid_up, jnp.zeros((pad,), jnp.int32)])
    src2 = src.reshape(NCHUNK, CH)
    dst2 = dst.reshape(NCHUNK, CH)
    kt2 = kt.reshape(NCHUNK, CH)
    ku2 = ku.reshape(NCHUNK, CH)

    y1 = _per_offset_transform(x, W_trans).reshape(K_OFF * N, C)
    p1 = _edge_accum(y1, src2, dst2, kt2)
    u = _lrelu_bn(p1, g_tbn, b_tbn)
    y2 = _per_offset_transform(u, W_up).reshape(K_OFF * N, C)
    p2 = _edge_accum(y2, src2, dst2, ku2)
    return _skip_bn(p2, skip, g3, b3)
